# baseline (speedup 1.0000x reference)
import functools

import jax
import jax.numpy as jnp
from jax import lax
from jax.experimental import pallas as pl
from jax.experimental.pallas import tpu as pltpu
from jax.experimental.pallas import tpu_sc as plsc

_TOP_K = 8
_NUM_EXPERTS = 64
_EPS = 1e-20
_TEMP = 1.0
_TILE_K = 32768

_E_SC = 8
_E_TC = _NUM_EXPERTS - _E_SC
_NC = 2
_NS = 16
_NW = _NC * _NS
_SC_CHUNK = 4096
_LANES = 16


def _tc_matmul_kernel(x_ref, w_ref, o_ref, acc_ref):
    k = pl.program_id(0)
    nk = pl.num_programs(0)

    @pl.when(k == 0)
    def _init():
        acc_ref[...] = jnp.zeros_like(acc_ref)

    acc_ref[...] += jax.lax.dot_general(
        x_ref[...], w_ref[...],
        dimension_numbers=(((1,), (1,)), ((), ())),
        preferred_element_type=jnp.float32)

    @pl.when(k == nk - 1)
    def _flush():
        o_ref[...] = acc_ref[...]


def _sc_body(B, K, x_hbm, w_hbm, out_hbm, xbuf, wbuf, accb):
    c = lax.axis_index("c")
    s = lax.axis_index("s")
    wid = s * _NC + c
    kw = K // _NW
    base = wid * kw
    nch = kw // _SC_CHUNK
    npairs = B * _E_SC
    nvreg = _SC_CHUNK // _LANES

    def chunk_step(i, _):
        off = base + i * _SC_CHUNK
        for b in range(B):
            pltpu.sync_copy(x_hbm.at[pl.ds(b * K + off, _SC_CHUNK)],
                            xbuf.at[b])
        pltpu.sync_copy(w_hbm.at[pl.ds(_E_TC, _E_SC), pl.ds(off, _SC_CHUNK)],
                        wbuf)

        def vbody(v, accs):
            o = v * _LANES
            xs = [xbuf[b, pl.ds(o, _LANES)] for b in range(B)]
            out = []
            for b in range(B):
                for e in range(_E_SC):
                    wv = wbuf[e, pl.ds(o, _LANES)]
                    out.append(accs[b * _E_SC + e] + wv * xs[b])
            return tuple(out)

        init = tuple(accb[pl.ds(j * _LANES, _LANES)] for j in range(npairs))
        accs = lax.fori_loop(0, nvreg, vbody, init)
        for j in range(npairs):
            accb[pl.ds(j * _LANES, _LANES)] = accs[j]
        return 0

    for j in range(npairs):
        accb[pl.ds(j * _LANES, _LANES)] = jnp.zeros((_LANES,), jnp.float32)
    lax.fori_loop(0, nch, chunk_step, 0)
    pltpu.sync_copy(accb, out_hbm.at[wid])


def _sc_partials(xf, W):
    B, K = xf.shape
    body = functools.partial(_sc_body, B, K)
    mesh = plsc.VectorSubcoreMesh(core_axis_name="c", subcore_axis_name="s")
    npairs = B * _E_SC
    f = pl.kernel(
        body, mesh=mesh,
        out_type=jax.ShapeDtypeStruct((_NW, npairs * _LANES), jnp.float32),
        scratch_types=[
            pltpu.VMEM((B, _SC_CHUNK), jnp.float32),
            pltpu.VMEM((_E_SC, _SC_CHUNK), jnp.float32),
            pltpu.VMEM((npairs * _LANES,), jnp.float32),
        ],
    )
    return f(xf.reshape(-1), W)


def _epilogue_kernel(tc_ref, sc_ref, bn_ref, o_ref):
    sc = sc_ref[...]
    logits_sc = jnp.sum(jnp.sum(sc, axis=3), axis=0)
    p = jnp.concatenate([tc_ref[...], logits_sc], axis=-1) + bn_ref[...]
    ps = p / _TEMP
    m = jnp.max(ps, axis=-1, keepdims=True)
    e = jnp.exp(ps - m)
    soft = e / jnp.sum(e, axis=-1, keepdims=True)
    t = jnp.max(p, axis=-1, keepdims=True)
    for _ in range(_TOP_K - 1):
        cnt = jnp.sum((p >= t).astype(jnp.int32), axis=-1, keepdims=True)
        nxt = jnp.max(jnp.where(p < t, p, -jnp.inf), axis=-1, keepdims=True)
        t = jnp.where(cnt >= _TOP_K, t, nxt)
    mask = jax.nn.sigmoid((p - t) / _TEMP)
    sm = soft * mask
    o_ref[...] = sm / jnp.sum(sm, axis=-1, keepdims=True)


def kernel(x, W, b):
    B = x.shape[0]
    E = _NUM_EXPERTS
    xf = x.reshape(B, -1)
    K = xf.shape[1]
    nk = K // _TILE_K
    U = jax.random.uniform(jax.random.key(1), (B, E), dtype=jnp.float32)
    noise = -jnp.log(-jnp.log(U + _EPS) + _EPS)
    bn = b[None, :] + noise

    sc_out = _sc_partials(xf, W)
    sc4d = sc_out.reshape(_NW, B, _E_SC, _LANES)

    tc_logits = pl.pallas_call(
        _tc_matmul_kernel,
        grid=(nk,),
        in_specs=[
            pl.BlockSpec((B, _TILE_K), lambda k: (0, k)),
            pl.BlockSpec((_E_TC, _TILE_K), lambda k: (0, k)),
        ],
        out_specs=pl.BlockSpec((B, _E_TC), lambda k: (0, 0)),
        out_shape=jax.ShapeDtypeStruct((B, _E_TC), jnp.float32),
        scratch_shapes=[pltpu.VMEM((B, _E_TC), jnp.float32)],
        compiler_params=pltpu.CompilerParams(
            dimension_semantics=("arbitrary",)),
    )(xf, W)

    return pl.pallas_call(
        _epilogue_kernel,
        in_specs=[
            pl.BlockSpec((B, _E_TC), lambda: (0, 0)),
            pl.BlockSpec((_NW, B, _E_SC, _LANES), lambda: (0, 0, 0, 0)),
            pl.BlockSpec((B, E), lambda: (0, 0)),
        ],
        out_specs=pl.BlockSpec((B, E), lambda: (0, 0)),
        out_shape=jax.ShapeDtypeStruct((B, E), jnp.float32),
    )(tc_logits, sc4d, bn)


# speedup vs baseline: 1.1643x; 1.1643x over previous
import functools

import jax
import jax.numpy as jnp
from jax import lax
from jax.experimental import pallas as pl
from jax.experimental.pallas import tpu as pltpu
from jax.experimental.pallas import tpu_sc as plsc

_TOP_K = 8
_NUM_EXPERTS = 64
_EPS = 1e-20
_TEMP = 1.0
_TILE_K = 32768

_E_SC = 8
_E_TC = _NUM_EXPERTS - _E_SC
_NC = 2
_NS = 16
_NW = _NC * _NS
_SC_CHUNK = 4096
_LANES = 16


def _tc_matmul_kernel(x_ref, w_ref, o_ref, acc_ref):
    k = pl.program_id(0)
    nk = pl.num_programs(0)

    @pl.when(k == 0)
    def _init():
        acc_ref[...] = jnp.zeros_like(acc_ref)

    xb = x_ref[...].reshape(x_ref.shape[0], -1)
    acc_ref[...] += jax.lax.dot_general(
        xb, w_ref[...],
        dimension_numbers=(((1,), (1,)), ((), ())),
        preferred_element_type=jnp.float32)

    @pl.when(k == nk - 1)
    def _flush():
        o_ref[...] = acc_ref[...]


def _sc_body(B, K, x_hbm, w_hbm, out_hbm, xbuf, wbuf, accb):
    c = lax.axis_index("c")
    s = lax.axis_index("s")
    wid = s * _NC + c
    kw = K // _NW
    base = wid * kw
    nch = kw // _SC_CHUNK
    npairs = B * _E_SC
    nvreg = _SC_CHUNK // _LANES

    def chunk_step(i, _):
        off = base + i * _SC_CHUNK
        pltpu.sync_copy(x_hbm.at[:, pl.ds(off, _SC_CHUNK)], xbuf)
        pltpu.sync_copy(w_hbm.at[pl.ds(_E_TC, _E_SC), pl.ds(off, _SC_CHUNK)],
                        wbuf)

        def vbody(v, accs):
            o = v * _LANES
            xs = [xbuf[b, pl.ds(o, _LANES)] for b in range(B)]
            out = []
            for b in range(B):
                for e in range(_E_SC):
                    wv = wbuf[e, pl.ds(o, _LANES)]
                    out.append(accs[b * _E_SC + e] + wv * xs[b])
            return tuple(out)

        init = tuple(accb[pl.ds(j * _LANES, _LANES)] for j in range(npairs))
        accs = lax.fori_loop(0, nvreg, vbody, init)
        for j in range(npairs):
            accb[pl.ds(j * _LANES, _LANES)] = accs[j]
        return 0

    for j in range(npairs):
        accb[pl.ds(j * _LANES, _LANES)] = jnp.zeros((_LANES,), jnp.float32)
    lax.fori_loop(0, nch, chunk_step, 0)
    pltpu.sync_copy(accb, out_hbm.at[wid])


def _sc_partials(xf, W):
    B, K = xf.shape
    body = functools.partial(_sc_body, B, K)
    mesh = plsc.VectorSubcoreMesh(core_axis_name="c", subcore_axis_name="s")
    npairs = B * _E_SC
    f = pl.kernel(
        body, mesh=mesh,
        out_type=jax.ShapeDtypeStruct((_NW, npairs * _LANES), jnp.float32),
        scratch_types=[
            pltpu.VMEM((B, _SC_CHUNK), jnp.float32),
            pltpu.VMEM((_E_SC, _SC_CHUNK), jnp.float32),
            pltpu.VMEM((npairs * _LANES,), jnp.float32),
        ],
    )
    return f(xf, W)


def _epilogue_kernel(tc_ref, sc_ref, bn_ref, o_ref):
    sc = sc_ref[...]
    logits_sc = jnp.sum(jnp.sum(sc, axis=3), axis=0)
    p = jnp.concatenate([tc_ref[...], logits_sc], axis=-1) + bn_ref[...]
    ps = p / _TEMP
    m = jnp.max(ps, axis=-1, keepdims=True)
    e = jnp.exp(ps - m)
    soft = e / jnp.sum(e, axis=-1, keepdims=True)
    t = jnp.max(p, axis=-1, keepdims=True)
    for _ in range(_TOP_K - 1):
        cnt = jnp.sum((p >= t).astype(jnp.int32), axis=-1, keepdims=True)
        nxt = jnp.max(jnp.where(p < t, p, -jnp.inf), axis=-1, keepdims=True)
        t = jnp.where(cnt >= _TOP_K, t, nxt)
    mask = jax.nn.sigmoid((p - t) / _TEMP)
    sm = soft * mask
    o_ref[...] = sm / jnp.sum(sm, axis=-1, keepdims=True)


def kernel(x, W, b):
    B = x.shape[0]
    E = _NUM_EXPERTS
    xf = x.reshape(B, -1)
    K = xf.shape[1]
    nk = K // _TILE_K
    U = jax.random.uniform(jax.random.key(1), (B, E), dtype=jnp.float32)
    noise = -jnp.log(-jnp.log(U + _EPS) + _EPS)
    bn = b[None, :] + noise

    sc_out = _sc_partials(xf, W)
    sc4d = sc_out.reshape(_NW, B, _E_SC, _LANES)

    ts = _TILE_K // x.shape[2]
    tc_logits = pl.pallas_call(
        _tc_matmul_kernel,
        grid=(nk,),
        in_specs=[
            pl.BlockSpec((B, ts, x.shape[2]), lambda k: (0, k, 0)),
            pl.BlockSpec((_E_TC, _TILE_K), lambda k: (0, k)),
        ],
        out_specs=pl.BlockSpec((B, _E_TC), lambda k: (0, 0)),
        out_shape=jax.ShapeDtypeStruct((B, _E_TC), jnp.float32),
        scratch_shapes=[pltpu.VMEM((B, _E_TC), jnp.float32)],
        compiler_params=pltpu.CompilerParams(
            dimension_semantics=("arbitrary",)),
    )(x, W)

    return pl.pallas_call(
        _epilogue_kernel,
        in_specs=[
            pl.BlockSpec((B, _E_TC), lambda: (0, 0)),
            pl.BlockSpec((_NW, B, _E_SC, _LANES), lambda: (0, 0, 0, 0)),
            pl.BlockSpec((B, E), lambda: (0, 0)),
        ],
        out_specs=pl.BlockSpec((B, E), lambda: (0, 0)),
        out_shape=jax.ShapeDtypeStruct((B, E), jnp.float32),
    )(tc_logits, sc4d, bn)


# speedup vs baseline: 1.3387x; 1.1498x over previous
import functools

import jax
import jax.numpy as jnp
from jax import lax
from jax.experimental import pallas as pl
from jax.experimental.pallas import tpu as pltpu
from jax.experimental.pallas import tpu_sc as plsc

_TOP_K = 8
_NUM_EXPERTS = 64
_EPS = 1e-20
_TEMP = 1.0
_TILE_K = 32768

_E_SC = 8
_E_TC = _NUM_EXPERTS - _E_SC
_NC = 2
_NS = 16
_NW = _NC * _NS
_SC_CHUNK = 4096
_LANES = 16


def _tc_matmul_kernel(x_ref, w_ref, o_ref, acc_ref):
    k = pl.program_id(0)
    nk = pl.num_programs(0)

    @pl.when(k == 0)
    def _init():
        acc_ref[...] = jnp.zeros_like(acc_ref)

    xb = x_ref[...].reshape(x_ref.shape[0], -1)
    acc_ref[...] += jax.lax.dot_general(
        xb, w_ref[...],
        dimension_numbers=(((1,), (1,)), ((), ())),
        preferred_element_type=jnp.float32)

    @pl.when(k == nk - 1)
    def _flush():
        o_ref[...] = acc_ref[...]


def _sc_body(B, S, H, x_hbm, w_hbm, out_hbm, xbuf, wbuf, accb):
    c = lax.axis_index("c")
    s = lax.axis_index("s")
    wid = s * _NC + c
    K = S * H
    kw = K // _NW
    sw = S // _NW
    spc = _SC_CHUNK // H
    base = wid * kw
    sbase = wid * sw
    nch = kw // _SC_CHUNK
    npairs = B * _E_SC
    nvh = H // _LANES

    def chunk_step(i, accs):
        koff = base + i * _SC_CHUNK
        soff = sbase + i * spc
        pltpu.sync_copy(x_hbm.at[:, pl.ds(soff, spc), :], xbuf)
        pltpu.sync_copy(w_hbm.at[pl.ds(_E_TC, _E_SC), pl.ds(koff, _SC_CHUNK)],
                        wbuf)

        for sl in range(spc):
            def vbody(v, a, sl=sl):
                o = v * _LANES
                xs = [xbuf[b, sl, pl.ds(o, _LANES)] for b in range(B)]
                out = []
                for b in range(B):
                    for e in range(_E_SC):
                        wv = wbuf[e, pl.ds(sl * H + o, _LANES)]
                        out.append(a[b * _E_SC + e] + wv * xs[b])
                return tuple(out)

            accs = lax.fori_loop(0, nvh, vbody, accs)
        return accs

    zero = jnp.zeros((_LANES,), jnp.float32)
    accs = lax.fori_loop(0, nch, chunk_step, (zero,) * npairs)
    for b in range(B):
        for e in range(_E_SC):
            accb[b, e, :] = accs[b * _E_SC + e]
    pltpu.sync_copy(accb, out_hbm.at[wid])


def _sc_partials(x, W):
    B, S, H = x.shape
    body = functools.partial(_sc_body, B, S, H)
    mesh = plsc.VectorSubcoreMesh(core_axis_name="c", subcore_axis_name="s")
    f = pl.kernel(
        body, mesh=mesh,
        out_type=jax.ShapeDtypeStruct((_NW, B, _E_SC, _LANES), jnp.float32),
        scratch_types=[
            pltpu.VMEM((B, _SC_CHUNK // H, H), jnp.float32),
            pltpu.VMEM((_E_SC, _SC_CHUNK), jnp.float32),
            pltpu.VMEM((B, _E_SC, _LANES), jnp.float32),
        ],
    )
    return f(x, W)


def _epilogue_kernel(tc_ref, sc_ref, bn_ref, o_ref):
    sc = sc_ref[...]
    logits_sc = jnp.sum(jnp.sum(sc, axis=3), axis=0)
    p = jnp.concatenate([tc_ref[...], logits_sc], axis=-1) + bn_ref[...]
    ps = p / _TEMP
    m = jnp.max(ps, axis=-1, keepdims=True)
    e = jnp.exp(ps - m)
    soft = e / jnp.sum(e, axis=-1, keepdims=True)
    t = jnp.max(p, axis=-1, keepdims=True)
    for _ in range(_TOP_K - 1):
        cnt = jnp.sum((p >= t).astype(jnp.int32), axis=-1, keepdims=True)
        nxt = jnp.max(jnp.where(p < t, p, -jnp.inf), axis=-1, keepdims=True)
        t = jnp.where(cnt >= _TOP_K, t, nxt)
    mask = jax.nn.sigmoid((p - t) / _TEMP)
    sm = soft * mask
    o_ref[...] = sm / jnp.sum(sm, axis=-1, keepdims=True)


def kernel(x, W, b):
    B = x.shape[0]
    E = _NUM_EXPERTS
    K = x.shape[1] * x.shape[2]
    nk = K // _TILE_K
    U = jax.random.uniform(jax.random.key(1), (B, E), dtype=jnp.float32)
    noise = -jnp.log(-jnp.log(U + _EPS) + _EPS)
    bn = b[None, :] + noise

    sc4d = _sc_partials(x, W)

    ts = _TILE_K // x.shape[2]
    tc_logits = pl.pallas_call(
        _tc_matmul_kernel,
        grid=(nk,),
        in_specs=[
            pl.BlockSpec((B, ts, x.shape[2]), lambda k: (0, k, 0)),
            pl.BlockSpec((_E_TC, _TILE_K), lambda k: (0, k)),
        ],
        out_specs=pl.BlockSpec((B, _E_TC), lambda k: (0, 0)),
        out_shape=jax.ShapeDtypeStruct((B, _E_TC), jnp.float32),
        scratch_shapes=[pltpu.VMEM((B, _E_TC), jnp.float32)],
        compiler_params=pltpu.CompilerParams(
            dimension_semantics=("arbitrary",)),
    )(x, W)

    return pl.pallas_call(
        _epilogue_kernel,
        in_specs=[
            pl.BlockSpec((B, _E_TC), lambda: (0, 0)),
            pl.BlockSpec((_NW, B, _E_SC, _LANES), lambda: (0, 0, 0, 0)),
            pl.BlockSpec((B, E), lambda: (0, 0)),
        ],
        out_specs=pl.BlockSpec((B, E), lambda: (0, 0)),
        out_shape=jax.ShapeDtypeStruct((B, E), jnp.float32),
    )(tc_logits, sc4d, bn)


# speedup vs baseline: 1.3402x; 1.0011x over previous
import functools

import jax
import jax.numpy as jnp
from jax import lax
from jax.experimental import pallas as pl
from jax.experimental.pallas import tpu as pltpu
from jax.experimental.pallas import tpu_sc as plsc

_TOP_K = 8
_NUM_EXPERTS = 64
_EPS = 1e-20
_TEMP = 1.0
_TILE_K = 32768

_E_SC = 8
_E_TC = _NUM_EXPERTS - _E_SC
_NC = 2
_NS = 16
_NW = _NC * _NS
_SC_CHUNK = 4096
_LANES = 16


def _tc_matmul_kernel(x_ref, w_ref, o_ref, acc_ref):
    k = pl.program_id(0)
    nk = pl.num_programs(0)

    @pl.when(k == 0)
    def _init():
        acc_ref[...] = jnp.zeros_like(acc_ref)

    xb = x_ref[...].reshape(x_ref.shape[0], -1)
    acc_ref[...] += jax.lax.dot_general(
        xb, w_ref[...],
        dimension_numbers=(((1,), (1,)), ((), ())),
        preferred_element_type=jnp.float32)

    @pl.when(k == nk - 1)
    def _flush():
        o_ref[...] = acc_ref[...]


def _sc_body(B, S, H, x_hbm, w_hbm, out_hbm, xbuf, wbuf, accb,
             sx0, sx1, sw0, sw1):
    c = lax.axis_index("c")
    s = lax.axis_index("s")
    wid = s * _NC + c
    K = S * H
    kw = K // _NW
    sw = S // _NW
    spc = _SC_CHUNK // H
    base = wid * kw
    sbase = wid * sw
    nch = kw // _SC_CHUNK
    npairs = B * _E_SC
    nvh = H // _LANES
    sems = ((sx0, sw0), (sx1, sw1))

    def copies(i, slot):
        koff = base + i * _SC_CHUNK
        soff = sbase + i * spc
        semx, semw = sems[slot]
        return (
            pltpu.make_async_copy(x_hbm.at[:, pl.ds(soff, spc), :],
                                  xbuf.at[slot], semx),
            pltpu.make_async_copy(
                w_hbm.at[pl.ds(_E_TC, _E_SC), pl.ds(koff, _SC_CHUNK)],
                wbuf.at[slot], semw),
        )

    def fire(i, slot):
        for cp in copies(i, slot):
            cp.start()

    def wait(i, slot):
        for cp in copies(i, slot):
            cp.wait()

    def compute(slot, accs):
        for sl in range(spc):
            def vbody(v, a, sl=sl):
                o = v * _LANES
                xs = [xbuf[slot, b, sl, pl.ds(o, _LANES)] for b in range(B)]
                out = []
                for b in range(B):
                    for e in range(_E_SC):
                        wv = wbuf[slot, e, pl.ds(sl * H + o, _LANES)]
                        out.append(a[b * _E_SC + e] + wv * xs[b])
                return tuple(out)

            accs = lax.fori_loop(0, nvh, vbody, accs)
        return accs

    fire(0, 0)
    npair_steps = nch // 2

    def pair_step(p, accs):
        i0 = 2 * p
        fire(i0 + 1, 1)
        wait(i0, 0)
        accs = compute(0, accs)

        @pl.when(p < npair_steps - 1)
        def _():
            fire(i0 + 2, 0)

        wait(i0 + 1, 1)
        accs = compute(1, accs)
        return accs

    zero = jnp.zeros((_LANES,), jnp.float32)
    accs = lax.fori_loop(0, npair_steps, pair_step, (zero,) * npairs)
    for b in range(B):
        for e in range(_E_SC):
            accb[b, e, :] = accs[b * _E_SC + e]
    pltpu.sync_copy(accb, out_hbm.at[wid])


def _sc_partials(x, W):
    B, S, H = x.shape
    body = functools.partial(_sc_body, B, S, H)
    mesh = plsc.VectorSubcoreMesh(core_axis_name="c", subcore_axis_name="s")
    f = pl.kernel(
        body, mesh=mesh,
        out_type=jax.ShapeDtypeStruct((_NW, B, _E_SC, _LANES), jnp.float32),
        scratch_types=[
            pltpu.VMEM((2, B, _SC_CHUNK // H, H), jnp.float32),
            pltpu.VMEM((2, _E_SC, _SC_CHUNK), jnp.float32),
            pltpu.VMEM((B, _E_SC, _LANES), jnp.float32),
            pltpu.SemaphoreType.DMA,
            pltpu.SemaphoreType.DMA,
            pltpu.SemaphoreType.DMA,
            pltpu.SemaphoreType.DMA,
        ],
    )
    return f(x, W)


def _epilogue_kernel(tc_ref, sc_ref, bn_ref, o_ref):
    sc = sc_ref[...]
    logits_sc = jnp.sum(jnp.sum(sc, axis=3), axis=0)
    p = jnp.concatenate([tc_ref[...], logits_sc], axis=-1) + bn_ref[...]
    ps = p / _TEMP
    m = jnp.max(ps, axis=-1, keepdims=True)
    e = jnp.exp(ps - m)
    soft = e / jnp.sum(e, axis=-1, keepdims=True)
    t = jnp.max(p, axis=-1, keepdims=True)
    for _ in range(_TOP_K - 1):
        cnt = jnp.sum((p >= t).astype(jnp.int32), axis=-1, keepdims=True)
        nxt = jnp.max(jnp.where(p < t, p, -jnp.inf), axis=-1, keepdims=True)
        t = jnp.where(cnt >= _TOP_K, t, nxt)
    mask = jax.nn.sigmoid((p - t) / _TEMP)
    sm = soft * mask
    o_ref[...] = sm / jnp.sum(sm, axis=-1, keepdims=True)


def kernel(x, W, b):
    B = x.shape[0]
    E = _NUM_EXPERTS
    K = x.shape[1] * x.shape[2]
    nk = K // _TILE_K
    U = jax.random.uniform(jax.random.key(1), (B, E), dtype=jnp.float32)
    noise = -jnp.log(-jnp.log(U + _EPS) + _EPS)
    bn = b[None, :] + noise

    sc4d = _sc_partials(x, W)

    ts = _TILE_K // x.shape[2]
    tc_logits = pl.pallas_call(
        _tc_matmul_kernel,
        grid=(nk,),
        in_specs=[
            pl.BlockSpec((B, ts, x.shape[2]), lambda k: (0, k, 0)),
            pl.BlockSpec((_E_TC, _TILE_K), lambda k: (0, k)),
        ],
        out_specs=pl.BlockSpec((B, _E_TC), lambda k: (0, 0)),
        out_shape=jax.ShapeDtypeStruct((B, _E_TC), jnp.float32),
        scratch_shapes=[pltpu.VMEM((B, _E_TC), jnp.float32)],
        compiler_params=pltpu.CompilerParams(
            dimension_semantics=("arbitrary",)),
    )(x, W)

    return pl.pallas_call(
        _epilogue_kernel,
        in_specs=[
            pl.BlockSpec((B, _E_TC), lambda: (0, 0)),
            pl.BlockSpec((_NW, B, _E_SC, _LANES), lambda: (0, 0, 0, 0)),
            pl.BlockSpec((B, E), lambda: (0, 0)),
        ],
        out_specs=pl.BlockSpec((B, E), lambda: (0, 0)),
        out_shape=jax.ShapeDtypeStruct((B, E), jnp.float32),
    )(tc_logits, sc4d, bn)
